# Initial kernel scaffold; baseline (speedup 1.0000x reference)
import jax
import jax.numpy as jnp
from jax.experimental import pallas as pl


def kernel(x, W_gate, W_noise):
    raise NotImplementedError("write your pallas kernel here")


# speedup vs baseline: 3.6304x; 3.6304x over previous
import functools

import jax
import jax.numpy as jnp
from jax.experimental import pallas as pl

N_TOK = 32768
D = 4096
E = 64
K = 8
BLK = 512

NEG_INF = float("-inf")


def _router_kernel(x_ref, w_ref, eps_ref, pw_ref, pi_ref):
    x = x_ref[...]
    w = w_ref[...]
    scores = jnp.dot(x, w, preferred_element_type=jnp.float32)
    gate = scores[:, :E]
    noise_std = jax.nn.softplus(scores[:, E:])
    s = gate + noise_std * eps_ref[...]

    iota = jax.lax.broadcasted_iota(jnp.int32, s.shape, 1)
    vals = []
    idxs = []
    cur = s
    for _ in range(K):
        m = jnp.max(cur, axis=1, keepdims=True)
        idx = jnp.min(jnp.where(cur == m, iota, E), axis=1, keepdims=True)
        vals.append(m)
        idxs.append(idx)
        cur = jnp.where(iota == idx, NEG_INF, cur)
    w8 = jnp.concatenate(vals, axis=1)
    i8 = jnp.concatenate(idxs, axis=1)
    e8 = jnp.exp(w8 - w8[:, :1])
    p8 = e8 / jnp.sum(e8, axis=1, keepdims=True)
    pw_ref[...] = p8
    pi_ref[...] = i8


@jax.jit
def _run(x, wcat_t, eps):
    grid = (N_TOK // BLK,)
    return pl.pallas_call(
        _router_kernel,
        grid=grid,
        in_specs=[
            pl.BlockSpec((BLK, D), lambda i: (i, 0)),
            pl.BlockSpec((D, 2 * E), lambda i: (0, 0)),
            pl.BlockSpec((BLK, E), lambda i: (i, 0)),
        ],
        out_specs=[
            pl.BlockSpec((BLK, K), lambda i: (i, 0)),
            pl.BlockSpec((BLK, K), lambda i: (i, 0)),
        ],
        out_shape=[
            jax.ShapeDtypeStruct((N_TOK, K), jnp.float32),
            jax.ShapeDtypeStruct((N_TOK, K), jnp.int32),
        ],
    )(x, wcat_t, eps)


def kernel(x, W_gate, W_noise):
    wcat_t = jnp.concatenate([W_gate, W_noise], axis=0).T
    eps = jax.random.normal(jax.random.key(42), (N_TOK, E), dtype=jnp.float32)
    return _run(x, wcat_t, eps)


# speedup vs baseline: 6.6877x; 1.8421x over previous
import functools

import jax
import jax.numpy as jnp
from jax.experimental import pallas as pl

N_TOK = 32768
D = 4096
E = 64
K = 8
BLK = 512

NEG_INF = float("-inf")


def _router_kernel(w_ref, x_ref, eps_ref, pw_ref, pi_ref):
    w = w_ref[...]
    x = x_ref[...]
    s2 = jax.lax.dot_general(
        w, x, (((1,), (1,)), ((), ())), preferred_element_type=jnp.float32
    )
    gate = s2[:E, :]
    noise_std = jax.nn.softplus(s2[E:, :])
    s = gate + noise_std * eps_ref[...]

    iota0 = jax.lax.broadcasted_iota(jnp.int32, (E, BLK), 0)
    vals = []
    idxs = []
    cur = s
    for _ in range(K):
        m = jnp.max(cur, axis=0, keepdims=True)
        idx = jnp.min(jnp.where(cur == m, iota0, E), axis=0, keepdims=True)
        vals.append(m)
        idxs.append(idx)
        cur = jnp.where(iota0 == idx, NEG_INF, cur)
    w8 = jnp.concatenate(vals, axis=0)
    i8 = jnp.concatenate(idxs, axis=0)
    e8 = jnp.exp(w8 - w8[0:1, :])
    p8 = e8 / jnp.sum(e8, axis=0, keepdims=True)
    pw_ref[...] = p8
    pi_ref[...] = i8


@jax.jit
def _run(x, wcat, eps_t):
    grid = (N_TOK // BLK,)
    pw_t, pi_t = pl.pallas_call(
        _router_kernel,
        grid=grid,
        in_specs=[
            pl.BlockSpec((2 * E, D), lambda i: (0, 0)),
            pl.BlockSpec((BLK, D), lambda i: (i, 0)),
            pl.BlockSpec((E, BLK), lambda i: (0, i)),
        ],
        out_specs=[
            pl.BlockSpec((K, BLK), lambda i: (0, i)),
            pl.BlockSpec((K, BLK), lambda i: (0, i)),
        ],
        out_shape=[
            jax.ShapeDtypeStruct((K, N_TOK), jnp.float32),
            jax.ShapeDtypeStruct((K, N_TOK), jnp.int32),
        ],
    )(wcat, x, eps_t)
    return pw_t.T, pi_t.T


def kernel(x, W_gate, W_noise):
    wcat = jnp.concatenate([W_gate, W_noise], axis=0)
    eps_t = jax.random.normal(jax.random.key(42), (N_TOK, E), dtype=jnp.float32).T
    return _run(x, wcat, eps_t)


# speedup vs baseline: 7.1529x; 1.0696x over previous
import functools

import jax
import jax.numpy as jnp
from jax.experimental import pallas as pl

N_TOK = 32768
D = 4096
E = 64
K = 8
BLK = 1024

NEG_INF = float("-inf")


def _router_kernel(w_ref, x_ref, eps_ref, pw_ref, pi_ref):
    w = w_ref[...]
    x = x_ref[...]
    s2 = jax.lax.dot_general(
        w, x, (((1,), (1,)), ((), ())), preferred_element_type=jnp.float32
    )
    gate = s2[:E, :]
    noise_std = jax.nn.softplus(s2[E:, :])
    s = gate + noise_std * eps_ref[...]

    iota0 = jax.lax.broadcasted_iota(jnp.int32, (E, BLK), 0)
    vals = []
    idxs = []
    cur = s
    for _ in range(K):
        m = jnp.max(cur, axis=0, keepdims=True)
        idx = jnp.min(jnp.where(cur == m, iota0, E), axis=0, keepdims=True)
        vals.append(m)
        idxs.append(idx)
        cur = jnp.where(iota0 == idx, NEG_INF, cur)
    w8 = jnp.concatenate(vals, axis=0)
    i8 = jnp.concatenate(idxs, axis=0)
    e8 = jnp.exp(w8 - w8[0:1, :])
    p8 = e8 / jnp.sum(e8, axis=0, keepdims=True)
    pw_ref[...] = p8
    pi_ref[...] = i8


@jax.jit
def _run(x, wcat, eps_t):
    grid = (N_TOK // BLK,)
    pw_t, pi_t = pl.pallas_call(
        _router_kernel,
        grid=grid,
        in_specs=[
            pl.BlockSpec((2 * E, D), lambda i: (0, 0)),
            pl.BlockSpec((BLK, D), lambda i: (i, 0)),
            pl.BlockSpec((E, BLK), lambda i: (0, i)),
        ],
        out_specs=[
            pl.BlockSpec((K, BLK), lambda i: (0, i)),
            pl.BlockSpec((K, BLK), lambda i: (0, i)),
        ],
        out_shape=[
            jax.ShapeDtypeStruct((K, N_TOK), jnp.float32),
            jax.ShapeDtypeStruct((K, N_TOK), jnp.int32),
        ],
    )(wcat, x, eps_t)
    return pw_t.T, pi_t.T


def kernel(x, W_gate, W_noise):
    wcat = jnp.concatenate([W_gate, W_noise], axis=0)
    eps_t = jax.random.normal(jax.random.key(42), (N_TOK, E), dtype=jnp.float32).T
    return _run(x, wcat, eps_t)


# speedup vs baseline: 7.1563x; 1.0005x over previous
import functools

import jax
import jax.numpy as jnp
from jax.experimental import pallas as pl

N_TOK = 32768
D = 4096
E = 64
K = 8
BLK = 1024
DH = D // 2

NEG_INF = float("-inf")


def _router_kernel(w_ref, x1_ref, x2_ref, eps_ref, pw_ref, pi_ref):
    w = w_ref[...]
    s2 = jax.lax.dot_general(
        w[:, :DH], x1_ref[...], (((1,), (1,)), ((), ())),
        preferred_element_type=jnp.float32,
    ) + jax.lax.dot_general(
        w[:, DH:], x2_ref[...], (((1,), (1,)), ((), ())),
        preferred_element_type=jnp.float32,
    )
    gate = s2[:E, :]
    noise_std = jax.nn.softplus(s2[E:, :])
    s = gate + noise_std * eps_ref[...]

    iota0 = jax.lax.broadcasted_iota(jnp.int32, (E, BLK), 0)
    vals = []
    idxs = []
    cur = s
    for _ in range(K):
        m = jnp.max(cur, axis=0, keepdims=True)
        idx = jnp.min(jnp.where(cur == m, iota0, E), axis=0, keepdims=True)
        vals.append(m)
        idxs.append(idx)
        cur = jnp.where(iota0 == idx, NEG_INF, cur)
    w8 = jnp.concatenate(vals, axis=0)
    i8 = jnp.concatenate(idxs, axis=0)
    e8 = jnp.exp(w8 - w8[0:1, :])
    p8 = e8 / jnp.sum(e8, axis=0, keepdims=True)
    pw_ref[...] = p8
    pi_ref[...] = i8


@jax.jit
def _run(x, wcat, eps_t):
    grid = (N_TOK // BLK,)
    pw_t, pi_t = pl.pallas_call(
        _router_kernel,
        grid=grid,
        in_specs=[
            pl.BlockSpec((2 * E, D), lambda i: (0, 0)),
            pl.BlockSpec((BLK, DH), lambda i: (i, 0)),
            pl.BlockSpec((BLK, DH), lambda i: (i, 1)),
            pl.BlockSpec((E, BLK), lambda i: (0, i)),
        ],
        out_specs=[
            pl.BlockSpec((K, BLK), lambda i: (0, i)),
            pl.BlockSpec((K, BLK), lambda i: (0, i)),
        ],
        out_shape=[
            jax.ShapeDtypeStruct((K, N_TOK), jnp.float32),
            jax.ShapeDtypeStruct((K, N_TOK), jnp.int32),
        ],
    )(wcat, x, x, eps_t)
    return pw_t.T, pi_t.T


def kernel(x, W_gate, W_noise):
    wcat = jnp.concatenate([W_gate, W_noise], axis=0)
    eps_t = jax.random.normal(jax.random.key(42), (N_TOK, E), dtype=jnp.float32).T
    return _run(x, wcat, eps_t)
